# baseline (speedup 1.0000x reference)
import functools

import jax
import jax.numpy as jnp
from jax import lax
from jax.experimental import pallas as pl
from jax.experimental.pallas import tpu as pltpu
from jax.experimental.pallas import tpu_sc as plsc

VOCAB = 1000000
D = 64
B = 4096
L = 200
HID = 256
NCLS = 10

NC = 2
NS = 16
NW = NC * NS

LA = 128
LB = L - LA
BPW = B // NW
NBUF = 4

_INV_L = 1.0 / L


def _split_body(t0, t1, t2, o0, o1, o2):
    for t, o in ((t0, o0), (t1, o1), (t2, o2)):
        x = t[...]
        o[0] = x[:, 0:LA]
        o[1, :, 0:LB] = x[:, LA:L]


def _split_tokens(tokens_0, tokens_1, tokens_2):
    RB = 256
    grid = (B // RB,)
    tok_spec = pl.BlockSpec((RB, L), lambda i: (i, 0))
    out_spec = pl.BlockSpec((2, RB, 128), lambda i: (0, i, 0))
    shape = jax.ShapeDtypeStruct((2, B, 128), jnp.int32)
    return pl.pallas_call(
        _split_body,
        grid=grid,
        in_specs=[tok_spec] * 3,
        out_specs=[out_spec] * 3,
        out_shape=[shape] * 3,
    )(tokens_0, tokens_1, tokens_2)


_VC = 8192


def _relayout_body(x_ref, o_ref):
    x = x_ref[...]
    row = lax.broadcasted_iota(jnp.int32, (D, 128), 0)
    col = lax.broadcasted_iota(jnp.int32, (D, 128), 1)
    se = (col == 2 * row).astype(jnp.float32)
    so = (col == 2 * row + 1).astype(jnp.float32)
    dn = (((1,), (1,)), ((), ()))
    for t in range(_VC // 128):
        xt = x[:, 128 * t:128 * (t + 1)]
        te = lax.dot_general(se, xt, dn, precision=lax.Precision.HIGHEST,
                             preferred_element_type=jnp.float32)
        to = lax.dot_general(so, xt, dn, precision=lax.Precision.HIGHEST,
                             preferred_element_type=jnp.float32)
        m = jnp.concatenate([te, to], axis=1)
        o_ref[64 * t:64 * (t + 1), :] = m


def _relayout_table(emb):
    merged = pl.pallas_call(
        _relayout_body,
        grid=(-(-VOCAB // _VC),),
        in_specs=[pl.BlockSpec((D, _VC), lambda i: (0, i))],
        out_specs=pl.BlockSpec((_VC // 2, 128), lambda i: (i, 0)),
        out_shape=jax.ShapeDtypeStruct((VOCAB // 2, 2 * D), jnp.float32),
    )(emb.T)
    return merged.reshape(VOCAB, D)


def _pool_body(tok, table, out, idx_v, rows_v, pooled_v, *sems):
    wid = lax.axis_index("s") * NC + lax.axis_index("c")
    base = wid * BPW

    def gather(table, b, buf):
        pltpu.async_copy(table.at[idx_v.at[0, b]],
                         rows_v.at[buf, pl.ds(0, LA)], sems[buf])
        pltpu.async_copy(table.at[idx_v.at[1, b, pl.ds(0, LB)]],
                         rows_v.at[buf, pl.ds(LA, LB)], sems[buf])

    def wait(table, b, buf):
        pltpu.make_async_copy(table.at[idx_v.at[0, b]],
                              rows_v.at[buf, pl.ds(0, LA)], sems[buf]).wait()
        pltpu.make_async_copy(table.at[idx_v.at[1, b, pl.ds(0, LB)]],
                              rows_v.at[buf, pl.ds(LA, LB)], sems[buf]).wait()

    def accum_store(b, buf):
        rbuf = rows_v.at[buf]

        def body(k, carry):
            a0, a1, a2, a3 = carry
            i = k * 8
            for j in range(8):
                a0 = a0 + rbuf[i + j, 0:16]
                a1 = a1 + rbuf[i + j, 16:32]
                a2 = a2 + rbuf[i + j, 32:48]
                a3 = a3 + rbuf[i + j, 48:64]
            return (a0, a1, a2, a3)

        z = jnp.zeros((16,), jnp.float32)
        a0, a1, a2, a3 = lax.fori_loop(0, L // 8, body, (z, z, z, z))
        pooled_v[b, 0:16] = a0 * _INV_L
        pooled_v[b, 16:32] = a1 * _INV_L
        pooled_v[b, 32:48] = a2 * _INV_L
        pooled_v[b, 48:64] = a3 * _INV_L

    MAIN = BPW - NBUF
    pltpu.sync_copy(tok.at[:, pl.ds(base, BPW)], idx_v)
    for b in range(NBUF - 1):
        gather(table, b, b)

    def step(bb):
        for u in range(NBUF):
            b = bb + u
            gather(table, b + NBUF - 1, (u + NBUF - 1) % NBUF)
            wait(table, b, u)
            accum_store(b, u)

    pl.loop(0, MAIN, step=NBUF)(step)
    gather(table, BPW - 1, NBUF - 1)
    for u in range(NBUF):
        b = MAIN + u
        wait(table, b, u)
        accum_store(b, u)

    pltpu.sync_copy(pooled_v, out.at[pl.ds(base, BPW)])


def _pooled_means(tok, emb):
    mesh = plsc.VectorSubcoreMesh(core_axis_name="c", subcore_axis_name="s",
                                  num_cores=NC, num_subcores=NS)
    return pl.kernel(
        _pool_body,
        out_type=jax.ShapeDtypeStruct((B, D), jnp.float32),
        mesh=mesh,
        compiler_params=pltpu.CompilerParams(use_tc_tiling_on_sc=False),
        scratch_types=[
            pltpu.VMEM((2, BPW, 128), jnp.int32),
            pltpu.VMEM((NBUF, L, D), jnp.float32),
            pltpu.VMEM((BPW, D), jnp.float32),
        ] + [pltpu.SemaphoreType.DMA] * NBUF,
    )(tok, emb)


def _mlp_body(x0_ref, x1_ref, x2_ref, w1_ref, b1_ref, w2_ref, b2_ref, o_ref):
    h = (jnp.dot(x0_ref[...], w1_ref[0], preferred_element_type=jnp.float32) +
         jnp.dot(x1_ref[...], w1_ref[1], preferred_element_type=jnp.float32) +
         jnp.dot(x2_ref[...], w1_ref[2], preferred_element_type=jnp.float32))
    h = jnp.maximum(h + b1_ref[...], 0.0)
    o_ref[...] = jnp.dot(h, w2_ref[...],
                         preferred_element_type=jnp.float32) + b2_ref[...]


def _mlp(p0, p1, p2, W1, b1, W2, b2):
    BB = 1024
    grid = (B // BB,)
    w1p = W1.reshape(3, D, HID)
    x_spec = pl.BlockSpec((BB, D), lambda i: (i, 0))
    return pl.pallas_call(
        _mlp_body,
        grid=grid,
        in_specs=[
            x_spec, x_spec, x_spec,
            pl.BlockSpec((3, D, HID), lambda i: (0, 0, 0)),
            pl.BlockSpec((1, HID), lambda i: (0, 0)),
            pl.BlockSpec((HID, NCLS), lambda i: (0, 0)),
            pl.BlockSpec((1, NCLS), lambda i: (0, 0)),
        ],
        out_specs=pl.BlockSpec((BB, NCLS), lambda i: (i, 0)),
        out_shape=jax.ShapeDtypeStruct((B, NCLS), jnp.float32),
    )(p0, p1, p2, w1p, b1.reshape(1, HID), W2, b2.reshape(1, NCLS))


def kernel(tokens_0, tokens_1, tokens_2, emb_uni, emb_bi, emb_tri,
           W1, b1, W2, b2):
    tok0, tok1, tok2 = _split_tokens(tokens_0, tokens_1, tokens_2)
    p0 = _pooled_means(tok0, _relayout_table(emb_uni))
    p1 = _pooled_means(tok1, _relayout_table(emb_bi))
    p2 = _pooled_means(tok2, _relayout_table(emb_tri))
    return _mlp(p0, p1, p2, W1, b1, W2, b2)


# speedup vs baseline: 2.9072x; 2.9072x over previous
import functools

import jax
import jax.numpy as jnp
from jax import lax
from jax.experimental import pallas as pl
from jax.experimental.pallas import tpu as pltpu
from jax.experimental.pallas import tpu_sc as plsc

VOCAB = 1000000
D = 64
B = 4096
L = 200
HID = 256
NCLS = 10

NC = 2
NS = 16
NW = NC * NS

LA = 128
LB = L - LA
BPW = B // NW
NBUF = 4

_INV_L = 1.0 / L


def _split_body(t0, t1, t2, o0, o1, o2):
    for t, o in ((t0, o0), (t1, o1), (t2, o2)):
        x = t[...]
        o[0] = x[:, 0:LA]
        o[1, :, 0:LB] = x[:, LA:L]


def _split_tokens(tokens_0, tokens_1, tokens_2):
    RB = 256
    grid = (B // RB,)
    tok_spec = pl.BlockSpec((RB, L), lambda i: (i, 0))
    out_spec = pl.BlockSpec((2, RB, 128), lambda i: (0, i, 0))
    shape = jax.ShapeDtypeStruct((2, B, 128), jnp.int32)
    return pl.pallas_call(
        _split_body,
        grid=grid,
        in_specs=[tok_spec] * 3,
        out_specs=[out_spec] * 3,
        out_shape=[shape] * 3,
    )(tokens_0, tokens_1, tokens_2)


_VC = 8192


def _relayout_body(x_ref, o_ref):
    x = x_ref[...]
    row = lax.broadcasted_iota(jnp.int32, (D, 128), 0)
    col = lax.broadcasted_iota(jnp.int32, (D, 128), 1)
    se = (col == 2 * row).astype(jnp.float32)
    so = (col == 2 * row + 1).astype(jnp.float32)
    dn = (((1,), (1,)), ((), ()))
    for t in range(_VC // 128):
        xt = x[:, 128 * t:128 * (t + 1)]
        te = lax.dot_general(se, xt, dn, preferred_element_type=jnp.float32)
        to = lax.dot_general(so, xt, dn, preferred_element_type=jnp.float32)
        m = jnp.concatenate([te, to], axis=1)
        o_ref[64 * t:64 * (t + 1), :] = m


def _relayout_table(emb):
    merged = pl.pallas_call(
        _relayout_body,
        grid=(-(-VOCAB // _VC),),
        in_specs=[pl.BlockSpec((D, _VC), lambda i: (0, i))],
        out_specs=pl.BlockSpec((_VC // 2, 128), lambda i: (i, 0)),
        out_shape=jax.ShapeDtypeStruct((VOCAB // 2, 2 * D), jnp.float32),
    )(emb.T)
    return merged.reshape(VOCAB, D)


def _pool_body(tok, table, out, idx_v, rows_v, pooled_v, *sems):
    wid = lax.axis_index("s") * NC + lax.axis_index("c")
    base = wid * BPW

    def gather(table, b, buf):
        pltpu.async_copy(table.at[idx_v.at[0, b]],
                         rows_v.at[buf, pl.ds(0, LA)], sems[buf])
        pltpu.async_copy(table.at[idx_v.at[1, b, pl.ds(0, LB)]],
                         rows_v.at[buf, pl.ds(LA, LB)], sems[buf])

    def wait(table, b, buf):
        pltpu.make_async_copy(table.at[idx_v.at[0, b]],
                              rows_v.at[buf, pl.ds(0, LA)], sems[buf]).wait()
        pltpu.make_async_copy(table.at[idx_v.at[1, b, pl.ds(0, LB)]],
                              rows_v.at[buf, pl.ds(LA, LB)], sems[buf]).wait()

    def accum_store(b, buf):
        rbuf = rows_v.at[buf]

        def body(k, carry):
            a0, a1, a2, a3 = carry
            i = k * 8
            for j in range(8):
                a0 = a0 + rbuf[i + j, 0:16]
                a1 = a1 + rbuf[i + j, 16:32]
                a2 = a2 + rbuf[i + j, 32:48]
                a3 = a3 + rbuf[i + j, 48:64]
            return (a0, a1, a2, a3)

        z = jnp.zeros((16,), jnp.float32)
        a0, a1, a2, a3 = lax.fori_loop(0, L // 8, body, (z, z, z, z))
        pooled_v[b, 0:16] = a0 * _INV_L
        pooled_v[b, 16:32] = a1 * _INV_L
        pooled_v[b, 32:48] = a2 * _INV_L
        pooled_v[b, 48:64] = a3 * _INV_L

    MAIN = BPW - NBUF
    pltpu.sync_copy(tok.at[:, pl.ds(base, BPW)], idx_v)
    for b in range(NBUF - 1):
        gather(table, b, b)

    def step(bb):
        for u in range(NBUF):
            b = bb + u
            gather(table, b + NBUF - 1, (u + NBUF - 1) % NBUF)
            wait(table, b, u)
            accum_store(b, u)

    pl.loop(0, MAIN, step=NBUF)(step)
    gather(table, BPW - 1, NBUF - 1)
    for u in range(NBUF):
        b = MAIN + u
        wait(table, b, u)
        accum_store(b, u)

    pltpu.sync_copy(pooled_v, out.at[pl.ds(base, BPW)])


def _pooled_means(tok, emb):
    mesh = plsc.VectorSubcoreMesh(core_axis_name="c", subcore_axis_name="s",
                                  num_cores=NC, num_subcores=NS)
    return pl.kernel(
        _pool_body,
        out_type=jax.ShapeDtypeStruct((B, D), jnp.float32),
        mesh=mesh,
        compiler_params=pltpu.CompilerParams(use_tc_tiling_on_sc=False),
        scratch_types=[
            pltpu.VMEM((2, BPW, 128), jnp.int32),
            pltpu.VMEM((NBUF, L, D), jnp.float32),
            pltpu.VMEM((BPW, D), jnp.float32),
        ] + [pltpu.SemaphoreType.DMA] * NBUF,
    )(tok, emb)


def _mlp_body(x0_ref, x1_ref, x2_ref, w1_ref, b1_ref, w2_ref, b2_ref, o_ref):
    h = (jnp.dot(x0_ref[...], w1_ref[0], preferred_element_type=jnp.float32) +
         jnp.dot(x1_ref[...], w1_ref[1], preferred_element_type=jnp.float32) +
         jnp.dot(x2_ref[...], w1_ref[2], preferred_element_type=jnp.float32))
    h = jnp.maximum(h + b1_ref[...], 0.0)
    o_ref[...] = jnp.dot(h, w2_ref[...],
                         preferred_element_type=jnp.float32) + b2_ref[...]


def _mlp(p0, p1, p2, W1, b1, W2, b2):
    BB = 1024
    grid = (B // BB,)
    w1p = W1.reshape(3, D, HID)
    x_spec = pl.BlockSpec((BB, D), lambda i: (i, 0))
    return pl.pallas_call(
        _mlp_body,
        grid=grid,
        in_specs=[
            x_spec, x_spec, x_spec,
            pl.BlockSpec((3, D, HID), lambda i: (0, 0, 0)),
            pl.BlockSpec((1, HID), lambda i: (0, 0)),
            pl.BlockSpec((HID, NCLS), lambda i: (0, 0)),
            pl.BlockSpec((1, NCLS), lambda i: (0, 0)),
        ],
        out_specs=pl.BlockSpec((BB, NCLS), lambda i: (i, 0)),
        out_shape=jax.ShapeDtypeStruct((B, NCLS), jnp.float32),
    )(p0, p1, p2, w1p, b1.reshape(1, HID), W2, b2.reshape(1, NCLS))


def kernel(tokens_0, tokens_1, tokens_2, emb_uni, emb_bi, emb_tri,
           W1, b1, W2, b2):
    tok0, tok1, tok2 = _split_tokens(tokens_0, tokens_1, tokens_2)
    p0 = _pooled_means(tok0, _relayout_table(emb_uni))
    p1 = _pooled_means(tok1, _relayout_table(emb_bi))
    p2 = _pooled_means(tok2, _relayout_table(emb_tri))
    return _mlp(p0, p1, p2, W1, b1, W2, b2)
